# baseline (speedup 1.0000x reference)
import numpy as np
import jax
import jax.numpy as jnp
from jax import lax
from jax.experimental import pallas as pl
from jax.experimental.pallas import tpu as pltpu

_B = 4096
_T = 1024
_F = 8
_REMOVE = 51 * _B
_SUB = 32
_LANE = 128
_TC = 128
_GRID = _T // _TC

_K1 = np.uint32(1234)
_K2 = np.uint32(1234 ^ 0x1BD11BDA)
_ROT_A = (13, 15, 26, 6)
_ROT_B = (17, 29, 16, 24)
_TINY = np.float32(np.finfo(np.float32).tiny)
_ONE_BITS = np.uint32(0x3F800000)


def _rounds(v0, v1, rots):
    for r in rots:
        v0 = v0 + v1
        v1 = v0 ^ ((v1 << np.uint32(r)) | lax.shift_right_logical(v1, np.uint32(32 - r)))
    return v0, v1


def _threefry_bits(x2):
    v1 = x2 + _K1
    v0 = v1
    v1 = v0 ^ ((v1 << np.uint32(13)) | lax.shift_right_logical(v1, np.uint32(19)))
    v0, v1 = _rounds(v0, v1, _ROT_A[1:])
    v0, v1 = v0 + _K1, v1 + (_K2 + np.uint32(1))
    v0, v1 = _rounds(v0, v1, _ROT_B)
    v0, v1 = v0 + _K2, v1 + np.uint32(2)
    v0, v1 = _rounds(v0, v1, _ROT_A)
    v0, v1 = v0, v1 + (_K1 + np.uint32(3))
    v0, v1 = _rounds(v0, v1, _ROT_B)
    v0, v1 = v0 + _K1, v1 + (_K2 + np.uint32(4))
    v0, v1 = _rounds(v0, v1, _ROT_A)
    v0, v1 = v0 + _K2, v1 + np.uint32(5)
    return v0 ^ v1


def _sample_index_tmajor(logits):
    bplane = lax.broadcasted_iota(jnp.int32, (_T, _B), 1) * (_T * _F)
    tplane = lax.broadcasted_iota(jnp.int32, (_T, _B), 0) * _F
    common = (bplane + tplane).astype(jnp.uint32)

    best_val = None
    best_idx = None
    for f in range(_F):
        bits = _threefry_bits(common + np.uint32(f))
        fm1 = lax.bitcast_convert_type(
            lax.shift_right_logical(bits, np.uint32(9)) | _ONE_BITS,
            jnp.float32) - np.float32(1.0)
        u = jnp.maximum(_TINY, fm1 + _TINY)
        val = -jnp.log(-jnp.log(u)) + logits[f]
        if f == 0:
            best_val = val
            best_idx = jnp.zeros((_T, _B), jnp.int32)
        else:
            upd = val > best_val
            best_val = jnp.where(upd, val, best_val)
            best_idx = jnp.where(upd, np.int32(f), best_idx)
    return best_idx


def _ifs_kernel(idx_ref, px_ref, py_ref, w_ref, b_ref, op_ref,
                xs_ref, ys_ref, os_ref, xc_ref, yc_ref, pg_ref):
    idx = idx_ref[...]
    bit0 = (idx & 1) != 0
    bit1 = (idx & 2) != 0
    bit2 = (idx & 4) != 0

    def gather8(c):
        s01 = jnp.where(bit0, c[1], c[0])
        s23 = jnp.where(bit0, c[3], c[2])
        s45 = jnp.where(bit0, c[5], c[4])
        s67 = jnp.where(bit0, c[7], c[6])
        s0123 = jnp.where(bit1, s23, s01)
        s4567 = jnp.where(bit1, s67, s45)
        return jnp.where(bit2, s4567, s0123)

    pg_ref[0] = gather8([w_ref[f, 0, 0] for f in range(_F)])
    pg_ref[1] = gather8([w_ref[f, 0, 1] for f in range(_F)])
    pg_ref[2] = gather8([w_ref[f, 1, 0] for f in range(_F)])
    pg_ref[3] = gather8([w_ref[f, 1, 1] for f in range(_F)])
    pg_ref[4] = gather8([b_ref[f, 0, 0] for f in range(_F)])
    pg_ref[5] = gather8([b_ref[f, 1, 0] for f in range(_F)])
    os_ref[...] = gather8([op_ref[f] for f in range(_F)])

    @pl.when(pl.program_id(0) == 0)
    def _():
        xc_ref[...] = px_ref[...]
        yc_ref[...] = py_ref[...]

    def body(t, carry):
        x, y = carry
        xn = pg_ref[0, t] * x + pg_ref[1, t] * y + pg_ref[4, t]
        yn = pg_ref[2, t] * x + pg_ref[3, t] * y + pg_ref[5, t]
        xs_ref[t] = xn
        ys_ref[t] = yn
        return xn, yn

    xN, yN = jax.lax.fori_loop(0, _TC, body, (xc_ref[...], yc_ref[...]),
                               unroll=8)
    xc_ref[...] = xN
    yc_ref[...] = yN


def kernel(point, optimized_weights, optimized_biases, optimized_function_ops, code):
    probs = code / jnp.sum(code)
    logits = jnp.log(probs)

    index_t = _sample_index_tmajor(logits)
    idx = index_t.reshape(_T, _SUB, _LANE)
    px = point[:, 0, 0].reshape(_SUB, _LANE)
    py = point[:, 1, 0].reshape(_SUB, _LANE)

    xs, ys, os_ = pl.pallas_call(
        _ifs_kernel,
        grid=(_GRID,),
        in_specs=[
            pl.BlockSpec((_TC, _SUB, _LANE), lambda i: (i, 0, 0)),
            pl.BlockSpec((_SUB, _LANE), lambda i: (0, 0)),
            pl.BlockSpec((_SUB, _LANE), lambda i: (0, 0)),
            pl.BlockSpec(memory_space=pltpu.SMEM),
            pl.BlockSpec(memory_space=pltpu.SMEM),
            pl.BlockSpec(memory_space=pltpu.SMEM),
        ],
        out_specs=[
            pl.BlockSpec((_TC, _SUB, _LANE), lambda i: (i, 0, 0)),
            pl.BlockSpec((_TC, _SUB, _LANE), lambda i: (i, 0, 0)),
            pl.BlockSpec((_TC, _SUB, _LANE), lambda i: (i, 0, 0)),
        ],
        out_shape=[jax.ShapeDtypeStruct((_T, _SUB, _LANE), jnp.float32)] * 3,
        scratch_shapes=[
            pltpu.VMEM((_SUB, _LANE), jnp.float32),
            pltpu.VMEM((_SUB, _LANE), jnp.float32),
            pltpu.VMEM((6, _TC, _SUB, _LANE), jnp.float32),
        ],
        compiler_params=pltpu.CompilerParams(
            dimension_semantics=("arbitrary",),
        ),
    )(idx, px, py, optimized_weights, optimized_biases, optimized_function_ops)

    pts = jnp.stack(
        [xs.reshape(_T, _B), ys.reshape(_T, _B), os_.reshape(_T, _B)], axis=-1
    )
    return pts.reshape(_T * _B, 3)[_REMOVE:]


# speedup vs baseline: 1.1958x; 1.1958x over previous
import numpy as np
import jax
import jax.numpy as jnp
from jax import lax
from jax.experimental import pallas as pl
from jax.experimental.pallas import tpu as pltpu

_B = 4096
_T = 1024
_F = 8
_REMOVE = 51 * _B
_SUB = 32
_LANE = 128
_TC = 128
_GRID = _T // _TC

_K1 = np.uint32(1234)
_K2 = np.uint32(1234 ^ 0x1BD11BDA)
_ROT_A = (13, 15, 26, 6)
_ROT_B = (17, 29, 16, 24)
_TINY = np.float32(np.finfo(np.float32).tiny)
_ONE_BITS = np.uint32(0x3F800000)


def _rounds(v0, v1, rots):
    for r in rots:
        v0 = v0 + v1
        v1 = v0 ^ ((v1 << np.uint32(r)) | lax.shift_right_logical(v1, np.uint32(32 - r)))
    return v0, v1


def _threefry_bits(x2):
    v1 = x2 + _K1
    v0 = v1
    v1 = v0 ^ ((v1 << np.uint32(13)) | lax.shift_right_logical(v1, np.uint32(19)))
    v0, v1 = _rounds(v0, v1, _ROT_A[1:])
    v0, v1 = v0 + _K1, v1 + (_K2 + np.uint32(1))
    v0, v1 = _rounds(v0, v1, _ROT_B)
    v0, v1 = v0 + _K2, v1 + np.uint32(2)
    v0, v1 = _rounds(v0, v1, _ROT_A)
    v0, v1 = v0, v1 + (_K1 + np.uint32(3))
    v0, v1 = _rounds(v0, v1, _ROT_B)
    v0, v1 = v0 + _K1, v1 + (_K2 + np.uint32(4))
    v0, v1 = _rounds(v0, v1, _ROT_A)
    v0, v1 = v0 + _K2, v1 + np.uint32(5)
    return v0 ^ v1


def _sample_index_tmajor(logits):
    bplane = lax.broadcasted_iota(jnp.int32, (_T, _B), 1) * (_T * _F)
    tplane = lax.broadcasted_iota(jnp.int32, (_T, _B), 0) * _F
    common = (bplane + tplane).astype(jnp.uint32)

    best_val = None
    best_idx = None
    for f in range(_F):
        bits = _threefry_bits(common + np.uint32(f))
        fm1 = lax.bitcast_convert_type(
            lax.shift_right_logical(bits, np.uint32(9)) | _ONE_BITS,
            jnp.float32) - np.float32(1.0)
        u = jnp.maximum(_TINY, fm1 + _TINY)
        val = -jnp.log(-jnp.log(u)) + logits[f]
        if f == 0:
            best_val = val
            best_idx = jnp.zeros((_T, _B), jnp.int32)
        else:
            upd = val > best_val
            best_val = jnp.where(upd, val, best_val)
            best_idx = jnp.where(upd, np.int32(f), best_idx)
    return best_idx


def _ifs_kernel(idx_ref, px_ref, py_ref, w_ref, b_ref, op_ref,
                xs_ref, ys_ref, os_ref, xc_ref, yc_ref, pg_ref):
    idx = idx_ref[...]
    bit0 = (idx & 1) != 0
    bit1 = (idx & 2) != 0
    bit2 = (idx & 4) != 0

    def gather8(c):
        s01 = jnp.where(bit0, c[1], c[0])
        s23 = jnp.where(bit0, c[3], c[2])
        s45 = jnp.where(bit0, c[5], c[4])
        s67 = jnp.where(bit0, c[7], c[6])
        s0123 = jnp.where(bit1, s23, s01)
        s4567 = jnp.where(bit1, s67, s45)
        return jnp.where(bit2, s4567, s0123)

    pg_ref[0] = gather8([w_ref[f, 0, 0] for f in range(_F)])
    pg_ref[1] = gather8([w_ref[f, 0, 1] for f in range(_F)])
    pg_ref[2] = gather8([w_ref[f, 1, 0] for f in range(_F)])
    pg_ref[3] = gather8([w_ref[f, 1, 1] for f in range(_F)])
    pg_ref[4] = gather8([b_ref[f, 0, 0] for f in range(_F)])
    pg_ref[5] = gather8([b_ref[f, 1, 0] for f in range(_F)])
    os_ref[...] = gather8([op_ref[f] for f in range(_F)])

    @pl.when(pl.program_id(0) == 0)
    def _():
        xc_ref[...] = px_ref[...]
        yc_ref[...] = py_ref[...]

    def rb(v):
        bits = lax.bitcast_convert_type(v, jnp.uint32)
        bits = bits + np.uint32(0x7FFF) + ((bits >> np.uint32(16)) & np.uint32(1))
        return lax.bitcast_convert_type(bits & np.uint32(0xFFFF0000), jnp.float32)

    def body(t, carry):
        x, y = carry
        xr, yr = rb(x), rb(y)
        xn = pg_ref[0, t] * xr + pg_ref[1, t] * yr + pg_ref[4, t]
        yn = pg_ref[2, t] * xr + pg_ref[3, t] * yr + pg_ref[5, t]
        xs_ref[t] = xn
        ys_ref[t] = yn
        return xn, yn

    xN, yN = jax.lax.fori_loop(0, _TC, body, (xc_ref[...], yc_ref[...]),
                               unroll=8)
    xc_ref[...] = xN
    yc_ref[...] = yN


def kernel(point, optimized_weights, optimized_biases, optimized_function_ops, code):
    probs = code / jnp.sum(code)
    logits = jnp.log(probs)

    index_t = jax.random.categorical(
        jax.random.key(1234), logits, shape=(_B, _T)).T
    idx = index_t.reshape(_T, _SUB, _LANE)
    wb = lax.bitcast_convert_type(optimized_weights, jnp.uint32)
    wb = (wb + np.uint32(0x7FFF) + ((wb >> np.uint32(16)) & np.uint32(1)))
    w_rounded = lax.bitcast_convert_type(wb & np.uint32(0xFFFF0000), jnp.float32)
    px = point[:, 0, 0].reshape(_SUB, _LANE)
    py = point[:, 1, 0].reshape(_SUB, _LANE)

    xs, ys, os_ = pl.pallas_call(
        _ifs_kernel,
        grid=(_GRID,),
        in_specs=[
            pl.BlockSpec((_TC, _SUB, _LANE), lambda i: (i, 0, 0)),
            pl.BlockSpec((_SUB, _LANE), lambda i: (0, 0)),
            pl.BlockSpec((_SUB, _LANE), lambda i: (0, 0)),
            pl.BlockSpec(memory_space=pltpu.SMEM),
            pl.BlockSpec(memory_space=pltpu.SMEM),
            pl.BlockSpec(memory_space=pltpu.SMEM),
        ],
        out_specs=[
            pl.BlockSpec((_TC, _SUB, _LANE), lambda i: (i, 0, 0)),
            pl.BlockSpec((_TC, _SUB, _LANE), lambda i: (i, 0, 0)),
            pl.BlockSpec((_TC, _SUB, _LANE), lambda i: (i, 0, 0)),
        ],
        out_shape=[jax.ShapeDtypeStruct((_T, _SUB, _LANE), jnp.float32)] * 3,
        scratch_shapes=[
            pltpu.VMEM((_SUB, _LANE), jnp.float32),
            pltpu.VMEM((_SUB, _LANE), jnp.float32),
            pltpu.VMEM((6, _TC, _SUB, _LANE), jnp.float32),
        ],
        compiler_params=pltpu.CompilerParams(
            dimension_semantics=("arbitrary",),
        ),
    )(idx, px, py, w_rounded, optimized_biases, optimized_function_ops)

    pts = jnp.stack(
        [xs.reshape(_T, _B), ys.reshape(_T, _B), os_.reshape(_T, _B)], axis=-1
    )
    return pts.reshape(_T * _B, 3)[_REMOVE:]
